# Initial kernel scaffold; baseline (speedup 1.0000x reference)
import jax
import jax.numpy as jnp
from jax.experimental import pallas as pl


def kernel(prediction, target, W, K):
    raise NotImplementedError("write your pallas kernel here")


# speedup vs baseline: 1.0714x; 1.0714x over previous
import functools

import jax
import jax.numpy as jnp
from jax import lax
from jax.experimental import pallas as pl
from jax.experimental.pallas import tpu as pltpu
from jax.experimental.pallas import tpu_sc as plsc

B, C, D = 1024, 1000, 64
DP = 128
CP = 1024
NW = 32
RPW = B // NW
NEG = -1.0e30
BIG = 3.0e38
BLK = 256


def _sc_body(pred_hbm, w_hbm, j0_hbm, wj_hbm, pred_v, idx_v, rows_v, sem):
    wid = lax.axis_index("s") * 2 + lax.axis_index("c")
    base = wid * RPW
    pltpu.sync_copy(pred_hbm.at[pl.ds(base * CP, RPW * CP)], pred_v)

    lane = lax.iota(jnp.int32, 16)

    for g in range(2):
        row_off = (g * 16 + lane) * CP

        def col_step(k, carry):
            ms = list(carry[:4])
            is_ = list(carry[4:])
            for t in range(4):
                c = t * 256 + k
                v = plsc.load_gather(pred_v, [row_off + c])
                b = v > ms[t]
                ms[t] = jnp.where(b, v, ms[t])
                is_[t] = jnp.where(b, jnp.full((16,), 1, jnp.int32) * c,
                                   is_[t])
            return tuple(ms) + tuple(is_)

        init = tuple(jnp.full((16,), -3.0e38, jnp.float32) for _ in range(4)) \
            + tuple(jnp.zeros((16,), jnp.int32) for _ in range(4))
        carry = lax.fori_loop(0, 256, col_step, init)
        m, i = carry[0], carry[4]
        for t in range(1, 4):
            b = carry[t] > m
            m = jnp.where(b, carry[t], m)
            i = jnp.where(b, carry[4 + t], i)
        idx_v[pl.ds(g * 16, 16)] = i

    pltpu.async_copy(w_hbm.at[idx_v], rows_v, sem).wait()
    pltpu.sync_copy(idx_v, j0_hbm.at[pl.ds(base, RPW)])
    pltpu.sync_copy(rows_v, wj_hbm.at[pl.ds(base, RPW)])


@functools.lru_cache(maxsize=1)
def _sc_topk_gather():
    return pl.kernel(
        _sc_body,
        out_type=(
            jax.ShapeDtypeStruct((B,), jnp.int32),
            jax.ShapeDtypeStruct((B, DP), jnp.float32),
        ),
        mesh=plsc.VectorSubcoreMesh(core_axis_name="c", subcore_axis_name="s"),
        compiler_params=pltpu.CompilerParams(needs_layout_passes=False),
        scratch_types=[
            pltpu.VMEM((RPW * CP,), jnp.float32),
            pltpu.VMEM((NW,), jnp.int32),
            pltpu.VMEM((RPW, DP), jnp.float32),
            pltpu.SemaphoreType.DMA,
        ],
    )


def _tc_body(pred_ref, wt_ref, wj_ref, j0_ref, k_ref, out_ref):
    pred = pred_ref[...]
    y0 = jnp.max(pred, axis=1, keepdims=True)
    margins = y0 - pred
    wt = wt_ref[...]
    wj = wj_ref[...]
    g = jnp.dot(wj, wt, preferred_element_type=jnp.float32)
    nj = jnp.sum(wj * wj, axis=1, keepdims=True)
    nc = jnp.sum(wt * wt, axis=0, keepdims=True)
    d2 = jnp.maximum(nj + nc - 2.0 * g, 0.0)
    dist = jnp.sqrt(d2) * k_ref[0, 0]
    cols = lax.broadcasted_iota(jnp.int32, (BLK, CP), 1)
    is_j0 = cols == j0_ref[...]
    ratio = jnp.where(is_j0, BIG, margins / jnp.where(is_j0, 1.0, dist))
    out_ref[...] = jnp.min(ratio, axis=1, keepdims=True)


def _tc_ratios(pred_pad, wt, wj, j0_col, k_smem):
    return pl.pallas_call(
        _tc_body,
        grid=(B // BLK,),
        in_specs=[
            pl.BlockSpec((BLK, CP), lambda i: (i, 0)),
            pl.BlockSpec((DP, CP), lambda i: (0, 0)),
            pl.BlockSpec((BLK, DP), lambda i: (i, 0)),
            pl.BlockSpec((BLK, 1), lambda i: (i, 0)),
            pl.BlockSpec(memory_space=pltpu.SMEM),
        ],
        out_specs=pl.BlockSpec((BLK, 1), lambda i: (i, 0)),
        out_shape=jax.ShapeDtypeStruct((B, 1), jnp.float32),
    )(pred_pad, wt, wj, j0_col, k_smem)


@jax.jit
def kernel(prediction, target, W, K):
    del target
    pred_pad = jnp.pad(prediction, ((0, 0), (0, CP - C)), constant_values=NEG)
    w_pad = jnp.pad(W, ((0, 0), (0, DP - D)))
    j0, wj = _sc_topk_gather()(pred_pad.reshape(B * CP), w_pad)
    wt = jnp.pad(W.T, ((0, DP - D), (0, CP - C)))
    out = _tc_ratios(pred_pad, wt, wj, j0.reshape(B, 1),
                     K.reshape(1, 1))
    return out[:, 0]


# speedup vs baseline: 1.4390x; 1.3430x over previous
import functools

import jax
import jax.numpy as jnp
from jax import lax
from jax.experimental import pallas as pl
from jax.experimental.pallas import tpu as pltpu
from jax.experimental.pallas import tpu_sc as plsc

B, C, D = 1024, 1000, 64
DP = 128
NW = 32
RPW = B // NW
BIG = 3.0e38
BLK = 256

_CHUNK_OFFS = [16 * k for k in range(C // 16)] + [C - 16]


def _sc_body(pred_hbm, w_hbm, j0_hbm, wj_hbm, pred_v, idx_v, rows_v, sem):
    wid = lax.axis_index("s") * 2 + lax.axis_index("c")
    base = wid * RPW
    pltpu.sync_copy(pred_hbm.at[pl.ds(base * C, RPW * C)], pred_v)

    lane = lax.iota(jnp.int32, 16)

    def row_step(r, carry):
        jlo, jhi = carry
        row0 = r * C
        ms = [jnp.full((16,), -3.0e38, jnp.float32) for _ in range(4)]
        is_ = [jnp.zeros((16,), jnp.int32) for _ in range(4)]
        for n, off in enumerate(_CHUNK_OFFS):
            t = n % 4
            v = pred_v[pl.ds(row0 + off, 16)]
            ids = lane + off
            b = v > ms[t]
            ms[t] = jnp.where(b, v, ms[t])
            is_[t] = jnp.where(b, ids, is_[t])
        m, i = ms[0], is_[0]
        for t in range(1, 4):
            b = (ms[t] > m) | ((ms[t] == m) & (is_[t] < i))
            m = jnp.where(b, ms[t], m)
            i = jnp.where(b, is_[t], i)
        y = jnp.max(m)
        cand = jnp.where(m == y, i, jnp.full((16,), 2**30, jnp.int32))
        j = jnp.min(cand)
        jlo = jnp.where(lane == r, j, jlo)
        jhi = jnp.where(lane == (r - 16), j, jhi)
        return jlo, jhi

    jlo, jhi = lax.fori_loop(
        0, RPW, row_step,
        (jnp.zeros((16,), jnp.int32), jnp.zeros((16,), jnp.int32)))
    idx_v[pl.ds(0, 16)] = jlo
    idx_v[pl.ds(16, 16)] = jhi

    pltpu.async_copy(w_hbm.at[idx_v], rows_v, sem).wait()
    pltpu.sync_copy(idx_v, j0_hbm.at[pl.ds(base, RPW)])
    pltpu.sync_copy(rows_v, wj_hbm.at[pl.ds(base, RPW)])


@functools.lru_cache(maxsize=1)
def _sc_topk_gather():
    return pl.kernel(
        _sc_body,
        out_type=(
            jax.ShapeDtypeStruct((B,), jnp.int32),
            jax.ShapeDtypeStruct((B, DP), jnp.float32),
        ),
        mesh=plsc.VectorSubcoreMesh(core_axis_name="c", subcore_axis_name="s"),
        compiler_params=pltpu.CompilerParams(needs_layout_passes=False),
        scratch_types=[
            pltpu.VMEM((RPW * C,), jnp.float32),
            pltpu.VMEM((NW,), jnp.int32),
            pltpu.VMEM((RPW, DP), jnp.float32),
            pltpu.SemaphoreType.DMA,
        ],
    )


def _tc_body(pred_ref, wt_ref, wj_ref, j0_ref, k_ref, out_ref):
    pred = pred_ref[...]
    y0 = jnp.max(pred, axis=1, keepdims=True)
    margins = y0 - pred
    wt = wt_ref[...]
    wj = wj_ref[...]
    g = jnp.dot(wj, wt, preferred_element_type=jnp.float32)
    nj = jnp.sum(wj * wj, axis=1, keepdims=True)
    nc = jnp.sum(wt * wt, axis=0, keepdims=True)
    d2 = jnp.maximum(nj + nc - 2.0 * g, 0.0)
    dist = jnp.sqrt(d2) * k_ref[0, 0]
    cols = lax.broadcasted_iota(jnp.int32, (BLK, C), 1)
    is_j0 = cols == j0_ref[...]
    ratio = jnp.where(is_j0, BIG, margins / jnp.where(is_j0, 1.0, dist))
    out_ref[...] = jnp.min(ratio, axis=1, keepdims=True)


def _tc_ratios(pred, wt, wj, j0_col, k_smem):
    return pl.pallas_call(
        _tc_body,
        grid=(B // BLK,),
        in_specs=[
            pl.BlockSpec((BLK, C), lambda i: (i, 0)),
            pl.BlockSpec((DP, C), lambda i: (0, 0)),
            pl.BlockSpec((BLK, DP), lambda i: (i, 0)),
            pl.BlockSpec((BLK, 1), lambda i: (i, 0)),
            pl.BlockSpec(memory_space=pltpu.SMEM),
        ],
        out_specs=pl.BlockSpec((BLK, 1), lambda i: (i, 0)),
        out_shape=jax.ShapeDtypeStruct((B, 1), jnp.float32),
    )(pred, wt, wj, j0_col, k_smem)


@jax.jit
def kernel(prediction, target, W, K):
    del target
    w_pad = jnp.pad(W, ((0, 0), (0, DP - D)))
    j0, wj = _sc_topk_gather()(prediction.reshape(B * C), w_pad)
    wt = jnp.pad(W.T, ((0, DP - D), (0, 0)))
    out = _tc_ratios(prediction, wt, wj, j0.reshape(B, 1), K.reshape(1, 1))
    return out[:, 0]


# speedup vs baseline: 1.5408x; 1.0708x over previous
import functools

import jax
import jax.numpy as jnp
from jax import lax
from jax.experimental import pallas as pl
from jax.experimental.pallas import tpu as pltpu
from jax.experimental.pallas import tpu_sc as plsc

B, C, D = 1024, 1000, 64
DP = 128
NW = 32
RPW = B // NW
BIG = 3.0e38
BLK = 256

_CHUNK_OFFS = [16 * k for k in range(C // 16)] + [C - 16]


def _sc_body(pred_hbm, w_hbm, j0_hbm, wj_hbm, pred_v, idx_v, rows_v, sem):
    wid = lax.axis_index("s") * 2 + lax.axis_index("c")
    base = wid * RPW
    pltpu.sync_copy(pred_hbm.at[pl.ds(base, RPW)], pred_v)

    lane = lax.iota(jnp.int32, 16)

    def row_step(r, carry):
        jlo, jhi = carry
        ms = [jnp.full((16,), -3.0e38, jnp.float32) for _ in range(4)]
        is_ = [jnp.zeros((16,), jnp.int32) for _ in range(4)]
        for n, off in enumerate(_CHUNK_OFFS):
            t = n % 4
            v = pred_v[r, pl.ds(off, 16)]
            ids = lane + off
            b = v > ms[t]
            ms[t] = jnp.where(b, v, ms[t])
            is_[t] = jnp.where(b, ids, is_[t])
        m, i = ms[0], is_[0]
        for t in range(1, 4):
            b = (ms[t] > m) | ((ms[t] == m) & (is_[t] < i))
            m = jnp.where(b, ms[t], m)
            i = jnp.where(b, is_[t], i)
        y = jnp.max(m)
        cand = jnp.where(m == y, i, jnp.full((16,), 2**30, jnp.int32))
        j = jnp.min(cand)
        jlo = jnp.where(lane == r, j, jlo)
        jhi = jnp.where(lane == (r - 16), j, jhi)
        return jlo, jhi

    jlo, jhi = lax.fori_loop(
        0, RPW, row_step,
        (jnp.zeros((16,), jnp.int32), jnp.zeros((16,), jnp.int32)))
    idx_v[pl.ds(0, 16)] = jlo
    idx_v[pl.ds(16, 16)] = jhi

    pltpu.async_copy(w_hbm.at[idx_v], rows_v, sem).wait()
    pltpu.sync_copy(idx_v, j0_hbm.at[pl.ds(base, RPW)])
    pltpu.sync_copy(rows_v, wj_hbm.at[pl.ds(base, RPW)])


@functools.lru_cache(maxsize=1)
def _sc_topk_gather():
    return pl.kernel(
        _sc_body,
        out_type=(
            jax.ShapeDtypeStruct((B,), jnp.int32),
            jax.ShapeDtypeStruct((B, DP), jnp.float32),
        ),
        mesh=plsc.VectorSubcoreMesh(core_axis_name="c", subcore_axis_name="s"),
        compiler_params=pltpu.CompilerParams(needs_layout_passes=False),
        scratch_types=[
            pltpu.VMEM((RPW, C), jnp.float32),
            pltpu.VMEM((NW,), jnp.int32),
            pltpu.VMEM((RPW, DP), jnp.float32),
            pltpu.SemaphoreType.DMA,
        ],
    )


def _tc_body(pred_ref, wt_ref, wj_ref, j0_ref, k_ref, out_ref):
    pred = pred_ref[...]
    y0 = jnp.max(pred, axis=1, keepdims=True)
    margins = y0 - pred
    wt = wt_ref[...]
    wj = wj_ref[...]
    g = jnp.dot(wj, wt, preferred_element_type=jnp.float32)
    nj = jnp.sum(wj * wj, axis=1, keepdims=True)
    nc = jnp.sum(wt * wt, axis=0, keepdims=True)
    d2 = jnp.maximum(nj + nc - 2.0 * g, 0.0)
    dist = jnp.sqrt(d2) * k_ref[0, 0]
    cols = lax.broadcasted_iota(jnp.int32, (BLK, C), 1)
    is_j0 = cols == j0_ref[...]
    ratio = jnp.where(is_j0, BIG, margins / jnp.where(is_j0, 1.0, dist))
    out_ref[...] = jnp.min(ratio, axis=1, keepdims=True)


def _tc_ratios(pred, wt, wj, j0_col, k_smem):
    return pl.pallas_call(
        _tc_body,
        grid=(B // BLK,),
        in_specs=[
            pl.BlockSpec((BLK, C), lambda i: (i, 0)),
            pl.BlockSpec((DP, C), lambda i: (0, 0)),
            pl.BlockSpec((BLK, DP), lambda i: (i, 0)),
            pl.BlockSpec((BLK, 1), lambda i: (i, 0)),
            pl.BlockSpec(memory_space=pltpu.SMEM),
        ],
        out_specs=pl.BlockSpec((BLK, 1), lambda i: (i, 0)),
        out_shape=jax.ShapeDtypeStruct((B, 1), jnp.float32),
    )(pred, wt, wj, j0_col, k_smem)


@jax.jit
def kernel(prediction, target, W, K):
    del target
    w_pad = jnp.pad(W, ((0, 0), (0, DP - D)))
    j0, wj = _sc_topk_gather()(prediction, w_pad)
    wt = jnp.pad(W.T, ((0, DP - D), (0, 0)))
    out = _tc_ratios(prediction, wt, wj, j0.reshape(B, 1), K.reshape(1, 1))
    return out[:, 0]
